# baseline (speedup 1.0000x reference)
import functools

import jax
import jax.numpy as jnp
from jax import lax
from jax.experimental import pallas as pl
from jax.experimental.pallas import tpu as pltpu
from jax.experimental.pallas import tpu_sc as plsc

B, L, V, D = 4096, 200, 1000000, 64
LN = 16
NC, NS = 2, 16
NW = NC * NS
BPW = B // NW
C0, C1 = 104, 96
NACC = D // LN


_GATHER_DNUMS = lax.GatherDimensionNumbers(
    offset_dims=(), collapsed_slice_dims=(0,), start_index_map=(0,))


def _bcast_lane(vec, j):
    idx = jnp.full((LN, 1), j, dtype=jnp.int32)
    return lax.gather(vec, idx, dimension_numbers=_GATHER_DNUMS,
                      slice_sizes=(1,),
                      mode=lax.GatherScatterMode.PROMISE_IN_BOUNDS)


_mesh = plsc.VectorSubcoreMesh(core_axis_name="c", subcore_axis_name="s")


@functools.partial(
    pl.kernel,
    out_type=jax.ShapeDtypeStruct((B * D,), jnp.float32),
    mesh=_mesh,
    compiler_params=pltpu.CompilerParams(use_tc_tiling_on_sc=False),
    scratch_types=[
        pltpu.VMEM((BPW * L,), jnp.int32),
        pltpu.VMEM((BPW * L,), jnp.float32),
        pltpu.VMEM((L, D), jnp.float32),
        pltpu.VMEM((BPW * D,), jnp.float32),
        pltpu.SemaphoreType.DMA,
    ],
)
def _embed_bag(idx_hbm, w_hbm, emb_hbm, out_hbm, idx_v, w_v, rows_v, out_v, sem):
    wid = lax.axis_index("s") * NC + lax.axis_index("c")

    pltpu.sync_copy(idx_hbm.at[pl.ds(pl.multiple_of(wid * (BPW * L), 8), BPW * L)],
                    idx_v)
    pltpu.sync_copy(w_hbm.at[pl.ds(pl.multiple_of(wid * (BPW * L), 8), BPW * L)],
                    w_v)

    def body(b, carry):
        off = pl.multiple_of(b * L, 8)
        pltpu.async_copy(
            emb_hbm.at[idx_v.at[pl.ds(off, C0)]],
            rows_v.at[pl.ds(0, C0)], sem).wait()
        pltpu.async_copy(
            emb_hbm.at[idx_v.at[pl.ds(pl.multiple_of(off + C0, 8), C1)]],
            rows_v.at[pl.ds(C0, C1)], sem).wait()

        accs = [jnp.zeros((LN,), jnp.float32) for _ in range(NACC)]
        for c in range(L // LN):
            wv = w_v[pl.ds(pl.multiple_of(b * L + c * LN, 8), LN)]
            for j in range(LN):
                wb = _bcast_lane(wv, j)
                r = c * LN + j
                for k in range(NACC):
                    accs[k] = accs[k] + wb * rows_v[r, pl.ds(k * LN, LN)]
        wv = w_v[pl.ds(pl.multiple_of(b * L + L - LN, 8), LN)]
        for j in range(LN - (L % LN), LN):
            wb = _bcast_lane(wv, j)
            r = L - LN + j
            for k in range(NACC):
                accs[k] = accs[k] + wb * rows_v[r, pl.ds(k * LN, LN)]

        obase = pl.multiple_of(b * D, 8)
        for k in range(NACC):
            out_v[pl.ds(pl.multiple_of(obase + k * LN, 8), LN)] = accs[k]
        return carry

    lax.fori_loop(0, BPW, body, 0)
    pltpu.sync_copy(out_v,
                    out_hbm.at[pl.ds(pl.multiple_of(wid * (BPW * D), 8), BPW * D)])


def kernel(x, embedding):
    idx = x[:, :, 0].astype(jnp.int32).reshape(B * L)
    w = x[:, :, 1].reshape(B * L)
    return _embed_bag(idx, w, embedding).reshape(B, D)


# speedup vs baseline: 1.1315x; 1.1315x over previous
import functools

import jax
import jax.numpy as jnp
from jax import lax
from jax.experimental import pallas as pl
from jax.experimental.pallas import tpu as pltpu
from jax.experimental.pallas import tpu_sc as plsc

B, L, V, D = 4096, 200, 1000000, 64
LN = 16
NC, NS = 2, 16
NW = NC * NS
BPW = B // NW
C0, C1 = 104, 96
NACC = D // LN


_GATHER_DNUMS = lax.GatherDimensionNumbers(
    offset_dims=(), collapsed_slice_dims=(0,), start_index_map=(0,))


def _bcast_lane(vec, j):
    idx = jnp.full((LN, 1), j, dtype=jnp.int32)
    return lax.gather(vec, idx, dimension_numbers=_GATHER_DNUMS,
                      slice_sizes=(1,),
                      mode=lax.GatherScatterMode.PROMISE_IN_BOUNDS)


_mesh = plsc.VectorSubcoreMesh(core_axis_name="c", subcore_axis_name="s")


@functools.partial(
    pl.kernel,
    out_type=jax.ShapeDtypeStruct((B * D,), jnp.float32),
    mesh=_mesh,
    compiler_params=pltpu.CompilerParams(use_tc_tiling_on_sc=False),
    scratch_types=[
        pltpu.VMEM((BPW * L,), jnp.int32),
        pltpu.VMEM((BPW * L,), jnp.float32),
        pltpu.VMEM((2, L, D), jnp.float32),
        pltpu.VMEM((BPW * D,), jnp.float32),
        pltpu.SemaphoreType.DMA,
        pltpu.SemaphoreType.DMA,
    ],
)
def _embed_bag(idx_hbm, w_hbm, emb_hbm, out_hbm,
               idx_v, w_v, rows_v, out_v, sem0, sem1):
    wid = lax.axis_index("s") * NC + lax.axis_index("c")
    sems = (sem0, sem1)

    pltpu.sync_copy(idx_hbm.at[pl.ds(pl.multiple_of(wid * (BPW * L), 8), BPW * L)],
                    idx_v)
    pltpu.sync_copy(w_hbm.at[pl.ds(pl.multiple_of(wid * (BPW * L), 8), BPW * L)],
                    w_v)

    def start_gather(b, buf):
        off = pl.multiple_of(b * L, 8)
        pltpu.async_copy(
            emb_hbm.at[idx_v.at[pl.ds(off, C0)]],
            rows_v.at[buf, pl.ds(0, C0)], sems[buf])
        pltpu.async_copy(
            emb_hbm.at[idx_v.at[pl.ds(pl.multiple_of(off + C0, 8), C1)]],
            rows_v.at[buf, pl.ds(C0, C1)], sems[buf])

    def wait_gather(buf):
        pltpu.make_async_copy(emb_hbm.at[pl.ds(0, L)],
                              rows_v.at[buf], sems[buf]).wait()

    def compute(b, buf):
        accs = [jnp.zeros((LN,), jnp.float32) for _ in range(NACC)]
        for c in range(L // LN):
            wv = w_v[pl.ds(pl.multiple_of(b * L + c * LN, 8), LN)]
            for j in range(LN):
                wb = _bcast_lane(wv, j)
                r = c * LN + j
                for k in range(NACC):
                    accs[k] = accs[k] + wb * rows_v[buf, r, pl.ds(k * LN, LN)]
        wv = w_v[pl.ds(pl.multiple_of(b * L + L - LN, 8), LN)]
        for j in range(LN - (L % LN), LN):
            wb = _bcast_lane(wv, j)
            r = L - LN + j
            for k in range(NACC):
                accs[k] = accs[k] + wb * rows_v[buf, r, pl.ds(k * LN, LN)]

        obase = pl.multiple_of(b * D, 8)
        for k in range(NACC):
            out_v[pl.ds(pl.multiple_of(obase + k * LN, 8), LN)] = accs[k]

    start_gather(0, 0)
    start_gather(1, 1)

    def body2(i, carry):
        b0 = i * 2
        for buf in range(2):
            b = b0 + buf
            wait_gather(buf)
            compute(b, buf)
            nxt = b + 2

            @pl.when(nxt < BPW)
            def _():
                start_gather(nxt, buf)
        return carry

    lax.fori_loop(0, BPW // 2, body2, 0)

    pltpu.sync_copy(out_v,
                    out_hbm.at[pl.ds(pl.multiple_of(wid * (BPW * D), 8), BPW * D)])


def kernel(x, embedding):
    idx = x[:, :, 0].astype(jnp.int32).reshape(B * L)
    w = x[:, :, 1].reshape(B * L)
    return _embed_bag(idx, w, embedding).reshape(B, D)
